# initial kernel scaffold (unmeasured)
import jax
import jax.numpy as jnp
from jax import lax
from jax.experimental import pallas as pl
from jax.experimental.pallas import tpu as pltpu

N_DEV = 4


def kernel(x, router_W, route_idx, expert_W, shared_W):
    n_tok, d_model = x.shape
    e_per, _, d_h = expert_W.shape

    xb = x.astype(jnp.bfloat16)
    rwb = router_W.astype(jnp.bfloat16)
    ewb = expert_W.astype(jnp.bfloat16)
    swb = shared_W.astype(jnp.bfloat16)

    def body(x_ref, rw_ref, idx_ref, ew_ref, sw_ref, out_ref,
             comm_ref, send_sems, recv_sems):
        my = lax.axis_index("i")
        left = lax.rem(my + N_DEV - 1, N_DEV)
        right = lax.rem(my + 1, N_DEV)

        barrier_sem = pltpu.get_barrier_semaphore()
        for nbr in (left, right):
            pl.semaphore_signal(
                barrier_sem, inc=1,
                device_id=(nbr,), device_id_type=pl.DeviceIdType.MESH,
            )
        pl.semaphore_wait(barrier_sem, 2)

        xv = x_ref[...]
        idx = idx_ref[...]

        scores = jnp.dot(xv, rw_ref[...], preferred_element_type=jnp.float32)
        s_max = jnp.max(scores, axis=-1, keepdims=True)
        e_s = jnp.exp(scores - s_max)
        probs = e_s / jnp.sum(e_s, axis=-1, keepdims=True)
        eids = lax.broadcasted_iota(jnp.int32, scores.shape, 1)
        p_sel = jnp.sum(jnp.where(eids == idx, probs, 0.0),
                        axis=-1, keepdims=True)

        acc = jnp.dot(xv, sw_ref[...], preferred_element_type=jnp.float32)

        def add_chunk(acc, origin, w_at):
            for j in range(e_per):
                e_glob = origin * e_per + j
                coeff = jnp.where(idx == e_glob, p_sel, 0.0).astype(jnp.bfloat16)
                acc = acc + jnp.dot(xv * coeff, w_at(j),
                                    preferred_element_type=jnp.float32)
            return acc

        acc = add_chunk(acc, my, lambda j: ew_ref[j])

        comm_ref[0, ...] = ew_ref[...]
        for h in range(N_DEV - 1):
            send_slot = h % 2
            recv_slot = (h + 1) % 2
            rdma = pltpu.make_async_remote_copy(
                src_ref=comm_ref.at[send_slot],
                dst_ref=comm_ref.at[recv_slot],
                send_sem=send_sems.at[send_slot],
                recv_sem=recv_sems.at[recv_slot],
                device_id=(right,),
                device_id_type=pl.DeviceIdType.MESH,
            )
            rdma.start()
            rdma.wait()
            origin = lax.rem(my + N_DEV - (h + 1), N_DEV)
            acc = add_chunk(acc, origin, lambda j: comm_ref[recv_slot, j])

        out_ref[...] = acc

    return pl.pallas_call(
        body,
        out_shape=jax.ShapeDtypeStruct((n_tok, d_h), jnp.float32),
        in_specs=[pl.BlockSpec(memory_space=pltpu.VMEM)] * 5,
        out_specs=pl.BlockSpec(memory_space=pltpu.VMEM),
        scratch_shapes=[
            pltpu.VMEM((2, e_per, d_model, d_h), jnp.bfloat16),
            pltpu.SemaphoreType.DMA((2,)),
            pltpu.SemaphoreType.DMA((2,)),
        ],
        compiler_params=pltpu.CompilerParams(collective_id=0),
    )(xb, rwb, route_idx, ewb, swb)


# baseline (device time: 384668 ns/iter reference)
import jax
import jax.numpy as jnp
from jax import lax
from jax.experimental import pallas as pl
from jax.experimental.pallas import tpu as pltpu

N_DEV = 4


def kernel(x, router_W, route_idx, expert_W, shared_W):
    n_tok, d_model = x.shape
    e_per, _, d_h = expert_W.shape

    xb = x.astype(jnp.bfloat16)
    rwb = router_W.astype(jnp.bfloat16)
    ewb = expert_W.astype(jnp.bfloat16)
    swb = shared_W.astype(jnp.bfloat16)

    def body(x_ref, rw_ref, idx_ref, ew_ref, sw_ref, out_ref,
             comm_ref, send_sems, recv_sems, local_sem):
        my = lax.axis_index("i")
        left = lax.rem(my + N_DEV - 1, N_DEV)
        right = lax.rem(my + 1, N_DEV)

        load = pltpu.make_async_copy(ew_ref, comm_ref.at[0], local_sem)
        load.start()

        barrier_sem = pltpu.get_barrier_semaphore()
        for nbr in (left, right):
            pl.semaphore_signal(
                barrier_sem, inc=1,
                device_id=(nbr,), device_id_type=pl.DeviceIdType.MESH,
            )
        pl.semaphore_wait(barrier_sem, 2)

        xv = x_ref[...]
        idx = idx_ref[...]

        scores = jnp.dot(xv, rw_ref[...], preferred_element_type=jnp.float32)
        s_max = jnp.max(scores, axis=-1, keepdims=True)
        e_s = jnp.exp(scores - s_max)
        probs = e_s / jnp.sum(e_s, axis=-1, keepdims=True)
        eids = lax.broadcasted_iota(jnp.int32, scores.shape, 1)
        p_sel = jnp.sum(jnp.where(eids == idx, probs, 0.0),
                        axis=-1, keepdims=True)

        out_ref[...] = jnp.dot(
            xv, sw_ref[...], preferred_element_type=jnp.float32
        ).astype(jnp.bfloat16)

        def add_chunk(origin, w_at):
            for j in range(e_per):
                e_glob = origin * e_per + j
                coeff = jnp.where(idx == e_glob, p_sel, 0.0).astype(jnp.bfloat16)
                out_ref[...] = (out_ref[...] + jnp.dot(
                    xv * coeff, w_at(j), preferred_element_type=jnp.float32)
                ).astype(jnp.bfloat16)

        load.wait()
        add_chunk(my, lambda j: comm_ref[0, j])

        for h in range(N_DEV - 1):
            send_slot = h % 2
            recv_slot = (h + 1) % 2
            rdma = pltpu.make_async_remote_copy(
                src_ref=comm_ref.at[send_slot],
                dst_ref=comm_ref.at[recv_slot],
                send_sem=send_sems.at[send_slot],
                recv_sem=recv_sems.at[recv_slot],
                device_id=(right,),
                device_id_type=pl.DeviceIdType.MESH,
            )
            rdma.start()
            rdma.wait()
            origin = lax.rem(my + N_DEV - (h + 1), N_DEV)
            add_chunk(origin, lambda j: comm_ref[recv_slot, j])

    return pl.pallas_call(
        body,
        out_shape=jax.ShapeDtypeStruct((n_tok, d_h), jnp.bfloat16),
        in_specs=[
            pl.BlockSpec(memory_space=pltpu.VMEM),
            pl.BlockSpec(memory_space=pltpu.VMEM),
            pl.BlockSpec(memory_space=pltpu.VMEM),
            pl.BlockSpec(memory_space=pltpu.MemorySpace.HBM),
            pl.BlockSpec(memory_space=pltpu.VMEM),
        ],
        out_specs=pl.BlockSpec(memory_space=pltpu.VMEM),
        scratch_shapes=[
            pltpu.VMEM((2, e_per, d_model, d_h), jnp.bfloat16),
            pltpu.SemaphoreType.DMA((2,)),
            pltpu.SemaphoreType.DMA((2,)),
            pltpu.SemaphoreType.DMA,
        ],
        compiler_params=pltpu.CompilerParams(
            collective_id=0, vmem_limit_bytes=48 * 1024 * 1024),
    )(xb, rwb, route_idx, ewb, swb)


# device time: 193836 ns/iter; 1.9845x vs baseline; 1.9845x over previous
import jax
import jax.numpy as jnp
from jax import lax
from jax.experimental import pallas as pl
from jax.experimental.pallas import tpu as pltpu

N_DEV = 4

FROM_L = 0
FROM_R = 1
OPP = 2


def kernel(x, router_W, route_idx, expert_W, shared_W):
    n_tok, d_model = x.shape
    e_per, _, d_h = expert_W.shape
    e_half = e_per // 2

    xb = x.astype(jnp.bfloat16)
    rwb = router_W.astype(jnp.bfloat16)
    ewb = expert_W.astype(jnp.bfloat16)
    swb = shared_W.astype(jnp.bfloat16)

    def body(x_ref, rw_ref, idx_ref, ew_ref, sw_ref, out_ref,
             gather_ref, stream_ref, p_ref, send_sems, recv_sems,
             stream_sems):
        my = lax.axis_index("i")
        left = lax.rem(my + N_DEV - 1, N_DEV)
        right = lax.rem(my + 1, N_DEV)

        barrier_sem = pltpu.get_barrier_semaphore()
        for nbr in (left, right):
            pl.semaphore_signal(
                barrier_sem, inc=1,
                device_id=(nbr,), device_id_type=pl.DeviceIdType.MESH,
            )
        pl.semaphore_wait(barrier_sem, 2)

        rdma_r = pltpu.make_async_remote_copy(
            src_ref=ew_ref,
            dst_ref=gather_ref.at[FROM_L],
            send_sem=send_sems.at[0],
            recv_sem=recv_sems.at[0],
            device_id=(right,),
            device_id_type=pl.DeviceIdType.MESH,
        )
        rdma_l = pltpu.make_async_remote_copy(
            src_ref=ew_ref,
            dst_ref=gather_ref.at[FROM_R],
            send_sem=send_sems.at[1],
            recv_sem=recv_sems.at[1],
            device_id=(left,),
            device_id_type=pl.DeviceIdType.MESH,
        )
        rdma_r.start()
        rdma_l.start()

        idx = idx_ref[...]
        scores = jnp.dot(x_ref[...], rw_ref[...],
                         preferred_element_type=jnp.float32)
        s_max = jnp.max(scores, axis=-1, keepdims=True)
        e_s = jnp.exp(scores - s_max)
        probs = e_s / jnp.sum(e_s, axis=-1, keepdims=True)
        eids = lax.broadcasted_iota(jnp.int32, scores.shape, 1)
        p_ref[...] = jnp.sum(jnp.where(eids == idx, probs, 0.0),
                             axis=-1, keepdims=True)

        TB = 512
        NB = n_tok // TB

        sw = sw_ref[...]

        def shared_blk(b, c):
            sl = pl.ds(b * TB, TB)
            out_ref[sl, :] = jnp.dot(
                x_ref[sl, :], sw, preferred_element_type=jnp.float32
            ).astype(jnp.bfloat16)
            return c

        lax.fori_loop(0, NB, shared_blk, 0)

        def add_expert(e_glob, w):
            def blk(b, c):
                sl = pl.ds(b * TB, TB)
                coeff = jnp.where(idx_ref[sl, :] == e_glob, p_ref[sl, :],
                                  0.0).astype(jnp.bfloat16)
                out_ref[sl, :] = (out_ref[sl, :] + jnp.dot(
                    x_ref[sl, :] * coeff, w,
                    preferred_element_type=jnp.float32)).astype(jnp.bfloat16)
                return c

            lax.fori_loop(0, NB, blk, 0)

        def add_chunk(origin, slot):
            for j in range(e_per):
                add_expert(origin * e_per + j, gather_ref[slot, j])

        loads = [
            pltpu.make_async_copy(
                ew_ref.at[j], stream_ref.at[j % 2], stream_sems.at[j % 2])
            for j in range(e_per)
        ]
        loads[0].start()
        for j in range(e_per):
            if j + 1 < e_per:
                loads[j + 1].start()
            loads[j].wait()
            add_expert(my * e_per + j, stream_ref[j % 2])

        rdma_r.wait_recv()
        rdma_l.wait_recv()

        rdma2_l = pltpu.make_async_remote_copy(
            src_ref=gather_ref.at[FROM_R, pl.ds(0, e_half)],
            dst_ref=gather_ref.at[OPP, pl.ds(0, e_half)],
            send_sem=send_sems.at[2],
            recv_sem=recv_sems.at[2],
            device_id=(left,),
            device_id_type=pl.DeviceIdType.MESH,
        )
        rdma2_r = pltpu.make_async_remote_copy(
            src_ref=gather_ref.at[FROM_L, pl.ds(e_half, e_half)],
            dst_ref=gather_ref.at[OPP, pl.ds(e_half, e_half)],
            send_sem=send_sems.at[3],
            recv_sem=recv_sems.at[3],
            device_id=(right,),
            device_id_type=pl.DeviceIdType.MESH,
        )
        rdma2_l.start()
        rdma2_r.start()

        add_chunk(left, FROM_L)
        add_chunk(right, FROM_R)

        rdma2_l.wait_recv()
        rdma2_r.wait_recv()
        opp = lax.rem(my + 2, N_DEV)
        add_chunk(opp, OPP)

        rdma_r.wait_send()
        rdma_l.wait_send()
        rdma2_l.wait_send()
        rdma2_r.wait_send()

    return pl.pallas_call(
        body,
        out_shape=jax.ShapeDtypeStruct((n_tok, d_h), jnp.bfloat16),
        in_specs=[
            pl.BlockSpec(memory_space=pltpu.VMEM),
            pl.BlockSpec(memory_space=pltpu.VMEM),
            pl.BlockSpec(memory_space=pltpu.VMEM),
            pl.BlockSpec(memory_space=pltpu.MemorySpace.HBM),
            pl.BlockSpec(memory_space=pltpu.VMEM),
        ],
        out_specs=pl.BlockSpec(memory_space=pltpu.VMEM),
        scratch_shapes=[
            pltpu.VMEM((3, e_per, d_model, d_h), jnp.bfloat16),
            pltpu.VMEM((2, d_model, d_h), jnp.bfloat16),
            pltpu.VMEM((n_tok, 1), jnp.float32),
            pltpu.SemaphoreType.DMA((4,)),
            pltpu.SemaphoreType.DMA((4,)),
            pltpu.SemaphoreType.DMA((2,)),
        ],
        compiler_params=pltpu.CompilerParams(
            collective_id=0, vmem_limit_bytes=46 * 1024 * 1024),
    )(xb, rwb, route_idx, ewb, swb)


# device time: 181837 ns/iter; 2.1155x vs baseline; 1.0660x over previous
import jax
import jax.numpy as jnp
from jax import lax
from jax.experimental import pallas as pl
from jax.experimental.pallas import tpu as pltpu

N_DEV = 4

FROM_L = 0
FROM_R = 1
OPP = 2


def kernel(x, router_W, route_idx, expert_W, shared_W):
    n_tok, d_model = x.shape
    e_per, _, d_h = expert_W.shape
    e_half = e_per // 2

    xb = x.astype(jnp.bfloat16)
    rwb = router_W.astype(jnp.bfloat16)
    ewb = expert_W.astype(jnp.bfloat16)
    swb = shared_W.astype(jnp.bfloat16)

    def body(x_ref, rw_ref, idx_ref, ew_ref, sw_ref, out_ref,
             gather_ref, stream_ref, p_ref, send_sems, recv_sems,
             stream_sems):
        my = lax.axis_index("i")
        left = lax.rem(my + N_DEV - 1, N_DEV)
        right = lax.rem(my + 1, N_DEV)

        barrier_sem = pltpu.get_barrier_semaphore()
        for nbr in (left, right):
            pl.semaphore_signal(
                barrier_sem, inc=1,
                device_id=(nbr,), device_id_type=pl.DeviceIdType.MESH,
            )
        pl.semaphore_wait(barrier_sem, 2)

        rdma_r = pltpu.make_async_remote_copy(
            src_ref=ew_ref,
            dst_ref=gather_ref.at[FROM_L],
            send_sem=send_sems.at[0],
            recv_sem=recv_sems.at[0],
            device_id=(right,),
            device_id_type=pl.DeviceIdType.MESH,
        )
        rdma_l = pltpu.make_async_remote_copy(
            src_ref=ew_ref,
            dst_ref=gather_ref.at[FROM_R],
            send_sem=send_sems.at[1],
            recv_sem=recv_sems.at[1],
            device_id=(left,),
            device_id_type=pl.DeviceIdType.MESH,
        )
        rdma_r.start()
        rdma_l.start()

        idx = idx_ref[...]
        scores = jnp.dot(x_ref[...], rw_ref[...],
                         preferred_element_type=jnp.float32)
        s_max = jnp.max(scores, axis=-1, keepdims=True)
        e_s = jnp.exp(scores - s_max)
        probs = e_s / jnp.sum(e_s, axis=-1, keepdims=True)
        eids = lax.broadcasted_iota(jnp.int32, scores.shape, 1)
        p_ref[...] = jnp.sum(jnp.where(eids == idx, probs, 0.0),
                             axis=-1, keepdims=True)

        TB = 512
        NB = n_tok // TB

        sw = sw_ref[...]

        def shared_blk(b, c):
            sl = pl.ds(b * TB, TB)
            out_ref[sl, :] = jnp.dot(
                x_ref[sl, :], sw, preferred_element_type=jnp.float32
            ).astype(jnp.bfloat16)
            return c

        lax.fori_loop(0, NB, shared_blk, 0)

        def add_expert(e_glob, w):
            def blk(b, c):
                sl = pl.ds(b * TB, TB)
                coeff = jnp.where(idx_ref[sl, :] == e_glob, p_ref[sl, :],
                                  0.0).astype(jnp.bfloat16)
                out_ref[sl, :] = (out_ref[sl, :] + jnp.dot(
                    x_ref[sl, :] * coeff, w,
                    preferred_element_type=jnp.float32)).astype(jnp.bfloat16)
                return c

            lax.fori_loop(0, NB, blk, 0)

        def add_chunk(origin, slot, j0=0, nj=None):
            nj = e_per if nj is None else nj

            def blk(b, c):
                sl = pl.ds(b * TB, TB)
                xb_blk = x_ref[sl, :]
                idx_blk = idx_ref[sl, :]
                p_blk = p_ref[sl, :]
                acc = out_ref[sl, :].astype(jnp.float32)
                for j in range(j0, j0 + nj):
                    e_glob = origin * e_per + j
                    coeff = jnp.where(idx_blk == e_glob, p_blk,
                                      0.0).astype(jnp.bfloat16)
                    acc = acc + jnp.dot(xb_blk * coeff, gather_ref[slot, j],
                                        preferred_element_type=jnp.float32)
                out_ref[sl, :] = acc.astype(jnp.bfloat16)
                return c

            lax.fori_loop(0, NB, blk, 0)

        loads = [
            pltpu.make_async_copy(
                ew_ref.at[j], stream_ref.at[j % 2], stream_sems.at[j % 2])
            for j in range(e_per)
        ]
        loads[0].start()
        for j in range(e_per):
            if j + 1 < e_per:
                loads[j + 1].start()
            loads[j].wait()
            add_expert(my * e_per + j, stream_ref[j % 2])

        rdma_r.wait_recv()
        rdma_l.wait_recv()

        rdma2_l = pltpu.make_async_remote_copy(
            src_ref=gather_ref.at[FROM_R, pl.ds(0, e_half)],
            dst_ref=gather_ref.at[OPP, pl.ds(0, e_half)],
            send_sem=send_sems.at[2],
            recv_sem=recv_sems.at[2],
            device_id=(left,),
            device_id_type=pl.DeviceIdType.MESH,
        )
        rdma2_r = pltpu.make_async_remote_copy(
            src_ref=gather_ref.at[FROM_L, pl.ds(e_half, e_half)],
            dst_ref=gather_ref.at[OPP, pl.ds(e_half, e_half)],
            send_sem=send_sems.at[3],
            recv_sem=recv_sems.at[3],
            device_id=(right,),
            device_id_type=pl.DeviceIdType.MESH,
        )
        rdma2_l.start()
        rdma2_r.start()

        add_chunk(left, FROM_L)
        add_chunk(right, FROM_R)

        opp = lax.rem(my + 2, N_DEV)
        rdma2_l.wait_recv()
        add_chunk(opp, OPP, 0, e_half)
        rdma2_r.wait_recv()
        add_chunk(opp, OPP, e_half, e_half)

        rdma_r.wait_send()
        rdma_l.wait_send()
        rdma2_l.wait_send()
        rdma2_r.wait_send()

    return pl.pallas_call(
        body,
        out_shape=jax.ShapeDtypeStruct((n_tok, d_h), jnp.bfloat16),
        in_specs=[
            pl.BlockSpec(memory_space=pltpu.VMEM),
            pl.BlockSpec(memory_space=pltpu.VMEM),
            pl.BlockSpec(memory_space=pltpu.VMEM),
            pl.BlockSpec(memory_space=pltpu.MemorySpace.HBM),
            pl.BlockSpec(memory_space=pltpu.VMEM),
        ],
        out_specs=pl.BlockSpec(memory_space=pltpu.VMEM),
        scratch_shapes=[
            pltpu.VMEM((3, e_per, d_model, d_h), jnp.bfloat16),
            pltpu.VMEM((2, d_model, d_h), jnp.bfloat16),
            pltpu.VMEM((n_tok, 1), jnp.float32),
            pltpu.SemaphoreType.DMA((4,)),
            pltpu.SemaphoreType.DMA((4,)),
            pltpu.SemaphoreType.DMA((2,)),
        ],
        compiler_params=pltpu.CompilerParams(
            collective_id=0, vmem_limit_bytes=46 * 1024 * 1024),
    )(xb, rwb, route_idx, ewb, swb)
